# baseline (speedup 1.0000x reference)
import functools

import jax
import jax.numpy as jnp
from jax import lax
from jax.experimental import pallas as pl
from jax.experimental.pallas import tpu as pltpu
from jax.experimental.pallas import tpu_sc as plsc

B = 16
H = 512
W = 512
C = 3
WC = W * C
PS = 240
PSC = PS * C
P = 64
PC = P * C
NB = 20
NBP = 32
MIN_PATCH_AREA = 60.0
SCALE = 0.3

CH = 16
NCH = (H // 2) // CH
SWW = 208

_MESH = dict(core_axis_name="c", subcore_axis_name="s",
             num_cores=2, num_subcores=16)
_LINEAR = pltpu.CompilerParams(use_tc_tiling_on_sc=False)


def _rng_consts():
    keys = jax.random.split(jax.random.key(42), B)
    ws, bs, noises = [], [], []
    for i in range(B):
        kw, kb, kn, kbr = jax.random.split(keys[i], 4)
        ws.append(jax.random.normal(kw, (1, 1, 3)) * 0.01 + 0.7)
        bs.append(jax.random.normal(kb, (1, 1, 3)) * 0.01 - 0.3)
        nz = jax.random.uniform(kn, (P, P, 3), minval=-0.1, maxval=0.1)
        br = jax.random.uniform(kbr, (), minval=-0.3, maxval=0.3)
        noises.append(nz + br)
    w = jnp.stack(ws).reshape(B, 3)
    b = jnp.stack(bs).reshape(B, 3)
    noise = jnp.stack(noises).reshape(B, P, PC)
    w_row = jnp.tile(w, (1, PS)).reshape(B, 1, PSC)
    b_row = jnp.tile(b, (1, PS)).reshape(B, 1, PSC)
    return w_row, b_row, noise


def _resize_mats():
    wm = jax.image.resize(jnp.eye(PS, dtype=jnp.float32), (P, PS),
                          method="linear")
    wit = jnp.einsum("jx,pq->xpjq", wm, jnp.eye(3, dtype=jnp.float32))
    return wm, wit.reshape(PSC, PC)


def _stamp_body(img_ref, patch_ref, w_ref, b_ref, noise_ref,
                b0_ref, b1_ref, b2_ref, b3_ref, wm_ref, wit_ref,
                im_ref, y0_ref, x0_ref, val_ref):
    mean_img = jnp.sum(img_ref[0]) / (H * W * C)
    p1 = jnp.clip(w_ref[0] * patch_ref[...] + b_ref[0], -1.0, 1.0)
    mean_p = jnp.sum(p1) / (PS * PS * C)
    p2 = jnp.clip(p1 + (mean_img - mean_p), -1.0, 1.0)
    r = jnp.dot(wm_ref[...], p2, preferred_element_type=jnp.float32)
    im = jnp.dot(r, wit_ref[...], preferred_element_type=jnp.float32)
    im_ref[0] = jnp.clip(im + noise_ref[0], -1.0, 1.0)
    a0, a1, a2, a3 = b0_ref[0], b1_ref[0], b2_ref[0], b3_ref[0]
    y0 = jnp.minimum(a0, a2) * float(H)
    y1 = jnp.maximum(a0, a2) * float(H)
    x0 = jnp.minimum(a1, a3) * float(W)
    x1 = jnp.maximum(a1, a3) * float(W)
    h = y1 - y0
    w = x1 - x0
    ps = jnp.floor(jnp.sqrt(h * w * SCALE))
    ymin = jnp.maximum(y0 + h * 0.5 - ps * 0.5, 0.0)
    xmin = jnp.maximum(x0 + w * 0.5 - ps * 0.5, 0.0)
    ymin = jnp.where(ymin + ps > float(H), float(H) - ps, ymin)
    xmin = jnp.where(xmin + ps > float(W), float(W) - ps, xmin)
    y0_ref[0] = jnp.clip(ymin.astype(jnp.int32), 0, H - P)
    x0_ref[0] = jnp.clip(xmin.astype(jnp.int32), 0, W - P)
    val_ref[0] = (ps * ps > MIN_PATCH_AREA).astype(jnp.int32)


def _stamps(images2, patch2, boxes):
    w_row, b_row, noise = _rng_consts()
    wm, wit = _resize_mats()
    bc = jnp.pad(boxes, ((0, 0), (0, NBP - NB), (0, 0)))
    bcs = [bc[:, :, k].reshape(B, 1, NBP) for k in range(4)]
    one = lambda i: (i, 0, 0)
    return pl.pallas_call(
        _stamp_body,
        grid=(B,),
        in_specs=[
            pl.BlockSpec((1, H, WC), one),
            pl.BlockSpec((PS, PSC), lambda i: (0, 0)),
            pl.BlockSpec((1, 1, PSC), one),
            pl.BlockSpec((1, 1, PSC), one),
            pl.BlockSpec((1, P, PC), one),
            pl.BlockSpec((1, 1, NBP), one),
            pl.BlockSpec((1, 1, NBP), one),
            pl.BlockSpec((1, 1, NBP), one),
            pl.BlockSpec((1, 1, NBP), one),
            pl.BlockSpec((P, PS), lambda i: (0, 0)),
            pl.BlockSpec((PSC, PC), lambda i: (0, 0)),
        ],
        out_specs=[
            pl.BlockSpec((1, P, PC), one),
            pl.BlockSpec((1, 1, NBP), one),
            pl.BlockSpec((1, 1, NBP), one),
            pl.BlockSpec((1, 1, NBP), one),
        ],
        out_shape=[
            jax.ShapeDtypeStruct((B, P, PC), jnp.float32),
            jax.ShapeDtypeStruct((B, 1, NBP), jnp.int32),
            jax.ShapeDtypeStruct((B, 1, NBP), jnp.int32),
            jax.ShapeDtypeStruct((B, 1, NBP), jnp.int32),
        ],
    )(images2, patch2, w_row, b_row, noise, *bcs, wm, wit)


def _sc_copy(images3):
    @functools.partial(
        pl.kernel,
        mesh=plsc.VectorSubcoreMesh(**_MESH),
        out_type=jax.ShapeDtypeStruct((B, H, WC), jnp.float32),
        compiler_params=_LINEAR,
        scratch_types=[
            pltpu.VMEM((CH, WC), jnp.float32),
            pltpu.VMEM((CH, WC), jnp.float32),
            pltpu.SemaphoreType.DMA,
            pltpu.SemaphoreType.DMA,
        ],
    )
    def k(img_hbm, out_hbm, bufa, bufb, rsem, wsem):
        c = lax.axis_index("c")
        s = lax.axis_index("s")
        b = c * 8 + s // 2
        r0 = (s % 2) * (H // 2)
        bufs = (bufa, bufb)
        reads = {}
        writes = {}
        reads[0] = pltpu.async_copy(
            img_hbm.at[b, pl.ds(r0, CH), :], bufs[0], rsem)
        for i in range(NCH):
            reads[i].wait()
            if i + 1 < NCH:
                if i >= 1:
                    writes[i - 1].wait()
                reads[i + 1] = pltpu.async_copy(
                    img_hbm.at[b, pl.ds(r0 + (i + 1) * CH, CH), :],
                    bufs[(i + 1) % 2], rsem)
            writes[i] = pltpu.async_copy(
                bufs[i % 2], out_hbm.at[b, pl.ds(r0 + i * CH, CH), :], wsem)
        if NCH >= 2:
            writes[NCH - 2].wait()
        writes[NCH - 1].wait()

    return k(images3)


def _sc_scatmask(out_ref0, images3, im, y0i, x0i, vali):
    @functools.partial(
        pl.kernel,
        mesh=plsc.VectorSubcoreMesh(**_MESH),
        out_type=jax.ShapeDtypeStruct((B, H, WC), jnp.float32),
        compiler_params=_LINEAR,
        scratch_types=[
            pltpu.VMEM((CH, WC), jnp.float32),
            pltpu.VMEM((CH, WC), jnp.float32),
            pltpu.VMEM((P, SWW), jnp.float32),
            pltpu.VMEM((P, PC), jnp.float32),
            pltpu.VMEM((NBP,), jnp.int32),
            pltpu.VMEM((NBP,), jnp.int32),
            pltpu.VMEM((NBP,), jnp.int32),
            pltpu.SemaphoreType.DMA,
            pltpu.SemaphoreType.DMA,
        ],
    )
    def k(out_hbm, img_hbm, im_hbm, y0_hbm, x0_hbm, val_hbm, mask_hbm,
          bufa, bufb, win, imb, yv, xv, vv, rsem, wsem):
        c = lax.axis_index("c")
        s = lax.axis_index("s")
        b = c * 8 + s // 2
        h = s % 2
        r0 = h * (H // 2)

        @pl.when(h == 0)
        def _():
            pltpu.sync_copy(im_hbm.at[b], imb)
            pltpu.sync_copy(y0_hbm.at[b], yv)
            pltpu.sync_copy(x0_hbm.at[b], xv)
            pltpu.sync_copy(val_hbm.at[b], vv)
            yva = yv[pl.ds(0, 16)]
            yvb = yv[pl.ds(16, 16)]
            xva = xv[pl.ds(0, 16)]
            xvb = xv[pl.ds(16, 16)]
            vva = vv[pl.ds(0, 16)]
            vvb = vv[pl.ds(16, 16)]
            for j in range(NB):
                lane = j % 16
                y0 = (yva if j < 16 else yvb)[lane]
                xc = (xva if j < 16 else xvb)[lane] * 3
                v = (vva if j < 16 else vvb)[lane]

                @pl.when(v == 1)
                def _(y0=y0, xc=xc):
                    wx = pl.multiple_of(
                        jnp.minimum((xc // 8) * 8, WC - SWW), 8)
                    dx = xc - wx
                    osl = (b, pl.ds(y0, P), pl.ds(wx, SWW))
                    pltpu.sync_copy(out_hbm.at[osl], win)

                    def ov(r, carry):
                        for kk in range(PC // 16):
                            win[r, pl.ds(dx + kk * 16, 16)] = (
                                imb[r, pl.ds(kk * 16, 16)])
                        return carry
                    lax.fori_loop(0, P, ov, 0)
                    pltpu.sync_copy(win, out_hbm.at[osl])

        plsc.subcore_barrier()

        def mchunk(ci, carry):
            rr = r0 + ci * CH
            ra = pltpu.async_copy(img_hbm.at[b, pl.ds(rr, CH), :], bufa, rsem)
            rb = pltpu.async_copy(out_hbm.at[b, pl.ds(rr, CH), :], bufb, wsem)
            ra.wait()
            rb.wait()

            def msub(t, carry2):
                row = t // 8
                base = (t % 8) * PC
                for kk in range(12):
                    sl = pl.ds(base + kk * 16, 16)
                    bufa[row, sl] = bufa[row, sl] - bufb[row, sl]
                return carry2
            lax.fori_loop(0, CH * 8, msub, 0)
            pltpu.sync_copy(bufa, mask_hbm.at[b, pl.ds(rr, CH), :])
            return carry
        lax.fori_loop(0, NCH, mchunk, 0)

    return k(out_ref0, images3, im, y0i, x0i, vali)


def kernel(boxes, images, patch):
    images2 = images.reshape(B, H, WC)
    patch2 = patch.reshape(PS, PSC)
    out0 = _sc_copy(images2)
    im, y0i, x0i, vali = _stamps(images2, patch2, boxes)
    oref = jax.new_ref(out0)
    mask = _sc_scatmask(oref, images2, im,
                        y0i.reshape(B, NBP), x0i.reshape(B, NBP),
                        vali.reshape(B, NBP))
    out = oref[...]
    return out.reshape(B, H, W, C), mask.reshape(B, H, W, C)


# speedup vs baseline: 1.0225x; 1.0225x over previous
import functools

import jax
import jax.numpy as jnp
from jax import lax
from jax.experimental import pallas as pl
from jax.experimental.pallas import tpu as pltpu
from jax.experimental.pallas import tpu_sc as plsc

B = 16
H = 512
W = 512
C = 3
WC = W * C
PS = 240
PSC = PS * C
P = 64
PC = P * C
NB = 20
NBP = 32
MIN_PATCH_AREA = 60.0
SCALE = 0.3

CH = 32
NCH = (H // 2) // CH
SWW = 208

_MESH = dict(core_axis_name="c", subcore_axis_name="s",
             num_cores=2, num_subcores=16)
_LINEAR = pltpu.CompilerParams(use_tc_tiling_on_sc=False)


def _rng_consts():
    keys = jax.random.split(jax.random.key(42), B)
    ws, bs, noises = [], [], []
    for i in range(B):
        kw, kb, kn, kbr = jax.random.split(keys[i], 4)
        ws.append(jax.random.normal(kw, (1, 1, 3)) * 0.01 + 0.7)
        bs.append(jax.random.normal(kb, (1, 1, 3)) * 0.01 - 0.3)
        nz = jax.random.uniform(kn, (P, P, 3), minval=-0.1, maxval=0.1)
        br = jax.random.uniform(kbr, (), minval=-0.3, maxval=0.3)
        noises.append(nz + br)
    w = jnp.stack(ws).reshape(B, 3)
    b = jnp.stack(bs).reshape(B, 3)
    noise = jnp.stack(noises).reshape(B, P, PC)
    w_row = jnp.tile(w, (1, PS)).reshape(B, 1, PSC)
    b_row = jnp.tile(b, (1, PS)).reshape(B, 1, PSC)
    return w_row, b_row, noise


def _resize_mats():
    wm = jax.image.resize(jnp.eye(PS, dtype=jnp.float32), (P, PS),
                          method="linear")
    wit = jnp.einsum("jx,pq->xpjq", wm, jnp.eye(3, dtype=jnp.float32))
    return wm, wit.reshape(PSC, PC)


def _stamp_body(img_ref, patch_ref, w_ref, b_ref, noise_ref,
                b0_ref, b1_ref, b2_ref, b3_ref, wm_ref, wit_ref,
                im_ref, y0_ref, x0_ref, val_ref):
    mean_img = jnp.sum(img_ref[0]) / (H * W * C)
    p1 = jnp.clip(w_ref[0] * patch_ref[...] + b_ref[0], -1.0, 1.0)
    mean_p = jnp.sum(p1) / (PS * PS * C)
    p2 = jnp.clip(p1 + (mean_img - mean_p), -1.0, 1.0)
    r = jnp.dot(wm_ref[...], p2, preferred_element_type=jnp.float32)
    im = jnp.dot(r, wit_ref[...], preferred_element_type=jnp.float32)
    im_ref[0] = jnp.clip(im + noise_ref[0], -1.0, 1.0)
    a0, a1, a2, a3 = b0_ref[0], b1_ref[0], b2_ref[0], b3_ref[0]
    y0 = jnp.minimum(a0, a2) * float(H)
    y1 = jnp.maximum(a0, a2) * float(H)
    x0 = jnp.minimum(a1, a3) * float(W)
    x1 = jnp.maximum(a1, a3) * float(W)
    h = y1 - y0
    w = x1 - x0
    ps = jnp.floor(jnp.sqrt(h * w * SCALE))
    ymin = jnp.maximum(y0 + h * 0.5 - ps * 0.5, 0.0)
    xmin = jnp.maximum(x0 + w * 0.5 - ps * 0.5, 0.0)
    ymin = jnp.where(ymin + ps > float(H), float(H) - ps, ymin)
    xmin = jnp.where(xmin + ps > float(W), float(W) - ps, xmin)
    y0_ref[0] = jnp.clip(ymin.astype(jnp.int32), 0, H - P)
    x0_ref[0] = jnp.clip(xmin.astype(jnp.int32), 0, W - P)
    val_ref[0] = (ps * ps > MIN_PATCH_AREA).astype(jnp.int32)


def _stamps(images2, patch2, boxes):
    w_row, b_row, noise = _rng_consts()
    wm, wit = _resize_mats()
    bc = jnp.pad(boxes, ((0, 0), (0, NBP - NB), (0, 0)))
    bcs = [bc[:, :, k].reshape(B, 1, NBP) for k in range(4)]
    one = lambda i: (i, 0, 0)
    return pl.pallas_call(
        _stamp_body,
        grid=(B,),
        in_specs=[
            pl.BlockSpec((1, H, WC), one),
            pl.BlockSpec((PS, PSC), lambda i: (0, 0)),
            pl.BlockSpec((1, 1, PSC), one),
            pl.BlockSpec((1, 1, PSC), one),
            pl.BlockSpec((1, P, PC), one),
            pl.BlockSpec((1, 1, NBP), one),
            pl.BlockSpec((1, 1, NBP), one),
            pl.BlockSpec((1, 1, NBP), one),
            pl.BlockSpec((1, 1, NBP), one),
            pl.BlockSpec((P, PS), lambda i: (0, 0)),
            pl.BlockSpec((PSC, PC), lambda i: (0, 0)),
        ],
        out_specs=[
            pl.BlockSpec((1, P, PC), one),
            pl.BlockSpec((1, 1, NBP), one),
            pl.BlockSpec((1, 1, NBP), one),
            pl.BlockSpec((1, 1, NBP), one),
        ],
        out_shape=[
            jax.ShapeDtypeStruct((B, P, PC), jnp.float32),
            jax.ShapeDtypeStruct((B, 1, NBP), jnp.int32),
            jax.ShapeDtypeStruct((B, 1, NBP), jnp.int32),
            jax.ShapeDtypeStruct((B, 1, NBP), jnp.int32),
        ],
    )(images2, patch2, w_row, b_row, noise, *bcs, wm, wit)


def _sc_scatter(images3, im, y0i, x0i, vali):
    @functools.partial(
        pl.kernel,
        mesh=plsc.VectorSubcoreMesh(**_MESH),
        out_type=[
            jax.ShapeDtypeStruct((B, H, WC), jnp.float32),
            jax.ShapeDtypeStruct((B, H, WC), jnp.float32),
        ],
        compiler_params=_LINEAR,
        scratch_types=[
            pltpu.VMEM((CH, WC), jnp.float32),
            pltpu.VMEM((CH, WC), jnp.float32),
            pltpu.VMEM((P, SWW), jnp.float32),
            pltpu.VMEM((P, PC), jnp.float32),
            pltpu.VMEM((NBP,), jnp.int32),
            pltpu.VMEM((NBP,), jnp.int32),
            pltpu.VMEM((NBP,), jnp.int32),
            pltpu.SemaphoreType.DMA,
            pltpu.SemaphoreType.DMA,
        ],
    )
    def k(img_hbm, im_hbm, y0_hbm, x0_hbm, val_hbm, out_hbm, mask_hbm,
          bufa, bufb, win, imb, yv, xv, vv, rsem, wsem):
        c = lax.axis_index("c")
        s = lax.axis_index("s")
        b = c * 8 + s // 2
        h = s % 2
        r0 = h * (H // 2)
        bufs = (bufa, bufb)

        reads = {}
        writes = {}
        reads[0] = pltpu.async_copy(
            img_hbm.at[b, pl.ds(r0, CH), :], bufs[0], rsem)
        for i in range(NCH):
            reads[i].wait()
            if i + 1 < NCH:
                if i >= 1:
                    writes[i - 1].wait()
                reads[i + 1] = pltpu.async_copy(
                    img_hbm.at[b, pl.ds(r0 + (i + 1) * CH, CH), :],
                    bufs[(i + 1) % 2], rsem)
            writes[i] = pltpu.async_copy(
                bufs[i % 2], out_hbm.at[b, pl.ds(r0 + i * CH, CH), :], wsem)
        if NCH >= 2:
            writes[NCH - 2].wait()
        writes[NCH - 1].wait()

        plsc.subcore_barrier()

        @pl.when(h == 0)
        def _():
            pltpu.sync_copy(im_hbm.at[b], imb)
            pltpu.sync_copy(y0_hbm.at[b], yv)
            pltpu.sync_copy(x0_hbm.at[b], xv)
            pltpu.sync_copy(val_hbm.at[b], vv)
            yva = yv[pl.ds(0, 16)]
            yvb = yv[pl.ds(16, 16)]
            xva = xv[pl.ds(0, 16)]
            xvb = xv[pl.ds(16, 16)]
            vva = vv[pl.ds(0, 16)]
            vvb = vv[pl.ds(16, 16)]
            for j in range(NB):
                lane = j % 16
                y0 = (yva if j < 16 else yvb)[lane]
                xc = (xva if j < 16 else xvb)[lane] * 3
                v = (vva if j < 16 else vvb)[lane]

                @pl.when(v == 1)
                def _(y0=y0, xc=xc):
                    wx = pl.multiple_of(
                        jnp.minimum((xc // 8) * 8, WC - SWW), 8)
                    dx = xc - wx
                    osl = (b, pl.ds(y0, P), pl.ds(wx, SWW))
                    pltpu.sync_copy(out_hbm.at[osl], win)

                    def ov(r, carry):
                        for kk in range(PC // 16):
                            win[r, pl.ds(dx + kk * 16, 16)] = (
                                imb[r, pl.ds(kk * 16, 16)])
                        return carry
                    lax.fori_loop(0, P, ov, 0)
                    pltpu.sync_copy(win, out_hbm.at[osl])

        plsc.subcore_barrier()

        def mchunk(ci, carry):
            rr = r0 + ci * CH
            ra = pltpu.async_copy(img_hbm.at[b, pl.ds(rr, CH), :], bufa, rsem)
            rb = pltpu.async_copy(out_hbm.at[b, pl.ds(rr, CH), :], bufb, wsem)
            ra.wait()
            rb.wait()

            def msub(t, carry2):
                row = t // 8
                base = (t % 8) * PC
                for kk in range(12):
                    sl = pl.ds(base + kk * 16, 16)
                    bufa[row, sl] = bufa[row, sl] - bufb[row, sl]
                return carry2
            lax.fori_loop(0, CH * 8, msub, 0)
            pltpu.sync_copy(bufa, mask_hbm.at[b, pl.ds(rr, CH), :])
            return carry
        lax.fori_loop(0, NCH, mchunk, 0)

    return k(images3, im, y0i, x0i, vali)


def kernel(boxes, images, patch):
    images2 = images.reshape(B, H, WC)
    patch2 = patch.reshape(PS, PSC)
    im, y0i, x0i, vali = _stamps(images2, patch2, boxes)
    out, mask = _sc_scatter(images2, im,
                            y0i.reshape(B, NBP), x0i.reshape(B, NBP),
                            vali.reshape(B, NBP))
    return out.reshape(B, H, W, C), mask.reshape(B, H, W, C)
